# Initial kernel scaffold; baseline (speedup 1.0000x reference)
import jax
import jax.numpy as jnp
from jax.experimental import pallas as pl


def kernel(x, edge_index, batch, y, params):
    raise NotImplementedError("write your pallas kernel here")


# speedup vs baseline: 27.7002x; 27.7002x over previous
import functools
import math

import jax
import jax.numpy as jnp
from jax import lax
from jax.experimental import pallas as pl
from jax.experimental.pallas import tpu as pltpu
from jax.experimental.pallas import tpu_sc as plsc

_N = 10000
_E = 320000
_G = 64
_HID = 64
_RATIO = 0.5
_NUM_LAYERS = 3

_NC = 2
_NS = 16
_NW = _NC * _NS
_NPAD = 10240
_NPER = _NPAD // _NS
_EPW = _E // _NW
_CHUNK = 1000
_NCHUNKS = _EPW // _CHUNK


def _make_segsum(C):
    mesh = plsc.VectorSubcoreMesh(core_axis_name="c", subcore_axis_name="s")

    @functools.partial(
        pl.kernel,
        mesh=mesh,
        compiler_params=pltpu.CompilerParams(use_tc_tiling_on_sc=False),
        out_type=jax.ShapeDtypeStruct((_NC * _NPAD, C), jnp.float32),
        scratch_types=[
            pltpu.VMEM((_CHUNK,), jnp.int32),
            pltpu.VMEM((_CHUNK,), jnp.int32),
            pltpu.VMEM((_CHUNK, C), jnp.float32),
            pltpu.VMEM_SHARED((_NPAD, C), jnp.float32),
            pltpu.SemaphoreType.DMA,
        ],
    )
    def seg(tbl_hbm, src_hbm, dst_hbm, zeros_hbm, out_hbm, srcv, dstv, rows, acc, sem):
        cid = lax.axis_index("c")
        sid = lax.axis_index("s")
        pltpu.sync_copy(
            zeros_hbm.at[pl.ds(sid * _NPER, _NPER)],
            acc.at[pl.ds(sid * _NPER, _NPER)],
        )
        plsc.subcore_barrier()
        wid = cid * _NS + sid
        for i in range(_NCHUNKS):
            base = wid * _EPW + i * _CHUNK
            pltpu.sync_copy(src_hbm.at[pl.ds(base, _CHUNK)], srcv)
            pltpu.sync_copy(dst_hbm.at[pl.ds(base, _CHUNK)], dstv)
            pltpu.async_copy(tbl_hbm.at[srcv], rows, sem).wait()
            pltpu.sync_copy(rows, acc.at[dstv], add=True)
        plsc.subcore_barrier()
        pltpu.sync_copy(
            acc.at[pl.ds(sid * _NPER, _NPER)],
            out_hbm.at[pl.ds(cid * _NPAD + sid * _NPER, _NPER)],
        )

    return seg


_seg64 = _make_segsum(_HID)
_seg8 = _make_segsum(8)


def _segsum(table, src, dst):
    C = table.shape[1]
    if C == _HID:
        out = _seg64(table, src, dst, jnp.zeros((_NPAD, C), jnp.float32))
        return out[:_N] + out[_NPAD:_NPAD + _N]
    tbl = jnp.concatenate([table, jnp.zeros((_N, 8 - C), jnp.float32)], axis=1)
    out = _seg8(tbl, src, dst, jnp.zeros((_NPAD, 8), jnp.float32))
    return (out[:_N] + out[_NPAD:_NPAD + _N])[:, :C]


def kernel(x, edge_index, batch, y, params):
    src = edge_index[0]
    dst = edge_index[1]
    mask = jnp.ones((_N,), jnp.float32)
    counts_all = jnp.bincount(batch, length=_G)
    starts = jnp.concatenate(
        [jnp.zeros((1,), counts_all.dtype), jnp.cumsum(counts_all)])[:_G]
    h_list = []
    xx = x.astype(jnp.float32)
    for step in range(_NUM_LAYERS):
        c = params["convs"][step]
        s = _segsum(mask[:, None], src, dst)[:, 0]
        deg = mask * s + mask
        dinv = jnp.where(deg > 0, 1.0 / jnp.sqrt(jnp.where(deg > 0, deg, 1.0)), 0.0)
        xw = xx @ c["W"]
        dm = dinv * mask
        agg_e = _segsum(dm[:, None] * xw, src, dst)
        agg = dm[:, None] * agg_e + (dinv * dinv * mask)[:, None] * xw
        xx = jax.nn.relu((agg + c["b"]) * mask[:, None])
        p = params["pools"][step]
        xn = xx @ p["Wn"]
        agg_s = _segsum(mask[:, None] * xn, src, dst)[:, 0]
        score = (xx @ p["Wr"])[:, 0] + mask * agg_s + p["b"][0]
        score_m = jnp.where(mask > 0, score, -jnp.inf)
        active = jax.ops.segment_sum(mask, batch, num_segments=_G)
        kcount = jnp.ceil(_RATIO * active)
        order = jnp.lexsort((-score_m, batch))
        pos = jnp.zeros((_N,), jnp.int32).at[order].set(jnp.arange(_N, dtype=jnp.int32))
        rank = pos - starts[batch].astype(jnp.int32)
        keep = (rank.astype(jnp.float32) < kcount[batch]) & (mask > 0)
        mask = keep.astype(jnp.float32)
        xx = xx * jnp.tanh(score)[:, None] * mask[:, None]
        xmax = jax.ops.segment_max(
            jnp.where(mask[:, None] > 0, xx, -jnp.inf), batch, num_segments=_G)
        xadd = jax.ops.segment_sum(xx, batch, num_segments=_G)
        h_list.append(jnp.concatenate([xmax, xadd], axis=1))
    hg = h_list[0] + h_list[1] + h_list[2]
    emb = hg @ params["lin"]["W"] + params["lin"]["b"]
    h = emb
    for li, layer in enumerate(params["mlp"]):
        h = h @ layer["W"] + layer["b"]
        if li < len(params["mlp"]) - 1:
            h = jax.nn.elu(h)
    out = h
    target = y[:, 0].reshape((_G, 1))
    loss = jnp.mean((out - target) ** 2)
    return out, loss
